# baseline (speedup 1.0000x reference)
import jax
import jax.numpy as jnp
from jax import lax
from jax.experimental import pallas as pl
from jax.experimental.pallas import tpu as pltpu
from jax.experimental.pallas import tpu_sc as plsc

B, T, V, D = 1024, 802, 133, 32
TP = 832
NC, NS = 2, 16
NW = NC * NS
TW = TP // NW
ROW = 4 * 8 * 8 * 128


def _sc_lookup(wflat, idxt, pe2):
    mesh = plsc.VectorSubcoreMesh(core_axis_name="c", subcore_axis_name="s")

    @pl.kernel(
        out_type=jax.ShapeDtypeStruct((T, ROW), jnp.float32),
        mesh=mesh,
        compiler_params=pltpu.CompilerParams(
            use_tc_tiling_on_sc=False, needs_layout_passes=False
        ),
        scratch_types=[
            pltpu.VMEM((V * D * 16,), jnp.float32),
            pltpu.VMEM((V * D,), jnp.float32),
            pltpu.VMEM((2, B), jnp.int32),
            pltpu.VMEM((ROW,), jnp.float32),
            pltpu.VMEM((2, D), jnp.float32),
            pltpu.SemaphoreType.DMA,
            pltpu.SemaphoreType.DMA,
            pltpu.SemaphoreType.DMA,
            pltpu.SemaphoreType.DMA,
        ],
    )
    def k(w_hbm, idxt_hbm, pe_hbm, out_hbm, rep_v, w_v, xcol_v, outv, pe_row,
          osem0, osem1, xsem, psem):
        wid = lax.axis_index("s") * NC + lax.axis_index("c")
        lane = lax.iota(jnp.int32, 16)
        pltpu.sync_copy(w_hbm, w_v)

        @plsc.parallel_loop(0, V * D // 16, unroll=2)
        def _(i):
            wv = w_v[pl.ds(i * 16, 16)]
            for j in range(16):
                rep_v[pl.ds((i * 16 + j) * 16, 16)] = jnp.broadcast_to(wv[j], (16,))

        HALF = ROW // 2
        osems = (osem0, osem1)

        def _wait_out(h):
            pltpu.make_async_copy(
                outv.at[pl.ds(0, HALF)], out_hbm.at[0, pl.ds(0, HALF)], osems[h]
            ).wait()

        def _fetch_in(t, slot):
            pltpu.async_copy(idxt_hbm.at[t], xcol_v.at[slot], xsem)
            pltpu.async_copy(pe_hbm.at[t], pe_row.at[slot], psem)

        def _wait_in(slot):
            pltpu.make_async_copy(pe_hbm.at[0], pe_row.at[slot], psem).wait()
            pltpu.make_async_copy(idxt_hbm.at[0], xcol_v.at[slot], xsem).wait()

        t0 = wid * TW

        @pl.when(t0 < T)
        def _():
            _fetch_in(t0, 0)

        @pl.loop(0, TW)
        def _(kk):
            t = t0 + kk

            @pl.when(t < T)
            def _():
                cur = kk % 2

                @pl.when((kk + 1 < TW) & (t + 1 < T))
                def _():
                    _fetch_in(t + 1, 1 - cur)

                _wait_in(cur)
                pe_lo = pe_row[cur, pl.ds(0, 16)]
                pe_hi = pe_row[cur, pl.ds(16, 16)]
                pes = [pe_lo[d] for d in range(16)] + [pe_hi[d] for d in range(16)]

                for h in range(2):
                    @pl.when(kk > 0)
                    def _():
                        _wait_out(h)

                    @plsc.parallel_loop(0, 64, unroll=8)
                    def _(c):
                        xv16 = (xcol_v[cur, pl.ds(c * 16, 16)] << 4) + lane
                        coff = (c // 8) * 1024 + (c % 8) * 16
                        for d in range(h * 16, h * 16 + 16):
                            g = plsc.load_gather(
                                rep_v.at[pl.ds(d * V * 16, V * 16)], [xv16]
                            )
                            off = ((d % 16) // 8) * 8192 + (d % 8) * 128
                            outv[pl.ds(h * HALF + coff + off, 16)] = g + pes[d]

                    pltpu.async_copy(
                        outv.at[pl.ds(h * HALF, HALF)],
                        out_hbm.at[t, pl.ds(h * HALF, HALF)],
                        osems[h],
                    )

        @pl.when(wid * TW < T)
        def _():
            _wait_out(0)
            _wait_out(1)

    return k(wflat, idxt, pe2)


def kernel(x, embed_weight, pe):
    x32 = x.astype(jnp.int32)
    idxt = jnp.transpose(x32)
    wflat = embed_weight.T.reshape(D * V)
    pe2 = pe.reshape(T, D)
    res = _sc_lookup(wflat, idxt, pe2)
    out4 = res.reshape(T, 4, 8, 8, 128)
    return out4.transpose(2, 4, 0, 1, 3).reshape(B, T, D)


# speedup vs baseline: 1.0036x; 1.0036x over previous
import jax
import jax.numpy as jnp
from jax import lax
from jax.experimental import pallas as pl
from jax.experimental.pallas import tpu as pltpu
from jax.experimental.pallas import tpu_sc as plsc

B, T, V, D = 1024, 802, 133, 32
TP = 832
NC, NS = 2, 16
NW = NC * NS
TW = TP // NW
ROW = 4 * 8 * 8 * 128


def _sc_lookup(wflat, idxt, pe2):
    mesh = plsc.VectorSubcoreMesh(core_axis_name="c", subcore_axis_name="s")

    @pl.kernel(
        out_type=jax.ShapeDtypeStruct((T, ROW), jnp.float32),
        mesh=mesh,
        compiler_params=pltpu.CompilerParams(
            use_tc_tiling_on_sc=False, needs_layout_passes=False
        ),
        scratch_types=[
            pltpu.VMEM((V * D * 16,), jnp.float32),
            pltpu.VMEM((V * D,), jnp.float32),
            pltpu.VMEM((2, B), jnp.int32),
            pltpu.VMEM((ROW,), jnp.float32),
            pltpu.VMEM((2, D), jnp.float32),
            pltpu.SemaphoreType.DMA,
            pltpu.SemaphoreType.DMA,
            pltpu.SemaphoreType.DMA,
            pltpu.SemaphoreType.DMA,
        ],
    )
    def k(w_hbm, idxt_hbm, pe_hbm, out_hbm, rep_v, w_v, xcol_v, outv, pe_row,
          osem0, osem1, xsem, psem):
        wid = lax.axis_index("s") * NC + lax.axis_index("c")
        lane = lax.iota(jnp.int32, 16)
        pltpu.sync_copy(w_hbm, w_v)

        @plsc.parallel_loop(0, V * D // 16, unroll=2)
        def _(i):
            wv = w_v[pl.ds(i * 16, 16)]
            for j in range(16):
                rep_v[pl.ds((i * 16 + j) * 16, 16)] = jnp.broadcast_to(wv[j], (16,))

        HALF = ROW // 2
        osems = (osem0, osem1)

        def _wait_out(h):
            pltpu.make_async_copy(
                outv.at[pl.ds(0, HALF)], out_hbm.at[0, pl.ds(0, HALF)], osems[h]
            ).wait()

        def _fetch_in(t, slot):
            pltpu.async_copy(idxt_hbm.at[t], xcol_v.at[slot], xsem)
            pltpu.async_copy(pe_hbm.at[t], pe_row.at[slot], psem)

        def _wait_in(slot):
            pltpu.make_async_copy(pe_hbm.at[0], pe_row.at[slot], psem).wait()
            pltpu.make_async_copy(idxt_hbm.at[0], xcol_v.at[slot], xsem).wait()

        t0 = wid * TW

        @pl.when(t0 < T)
        def _():
            _fetch_in(t0, 0)

        @pl.loop(0, TW)
        def _(kk):
            t = t0 + kk

            @pl.when(t < T)
            def _():
                cur = kk % 2

                @pl.when((kk + 1 < TW) & (t + 1 < T))
                def _():
                    _fetch_in(t + 1, 1 - cur)

                _wait_in(cur)
                pe_lo = pe_row[cur, pl.ds(0, 16)]
                pe_hi = pe_row[cur, pl.ds(16, 16)]
                pes = [pe_lo[d] for d in range(16)] + [pe_hi[d] for d in range(16)]

                for h in range(2):
                    @pl.when(kk > 0)
                    def _():
                        _wait_out(h)

                    @plsc.parallel_loop(0, 64, unroll=4)
                    def _(c):
                        xv16 = (xcol_v[cur, pl.ds(c * 16, 16)] << 4) + lane
                        coff = (c // 8) * 1024 + (c % 8) * 16
                        for d in range(h * 16, h * 16 + 16):
                            g = plsc.load_gather(
                                rep_v.at[pl.ds(d * V * 16, V * 16)], [xv16]
                            )
                            off = ((d % 16) // 8) * 8192 + (d % 8) * 128
                            outv[pl.ds(h * HALF + coff + off, 16)] = g + pes[d]

                    pltpu.async_copy(
                        outv.at[pl.ds(h * HALF, HALF)],
                        out_hbm.at[t, pl.ds(h * HALF, HALF)],
                        osems[h],
                    )

        @pl.when(wid * TW < T)
        def _():
            _wait_out(0)
            _wait_out(1)

    return k(wflat, idxt, pe2)


def kernel(x, embed_weight, pe):
    x32 = x.astype(jnp.int32)
    idxt = jnp.transpose(x32)
    wflat = embed_weight.T.reshape(D * V)
    pe2 = pe.reshape(T, D)
    res = _sc_lookup(wflat, idxt, pe2)
    out4 = res.reshape(T, 4, 8, 8, 128)
    return out4.transpose(2, 4, 0, 1, 3).reshape(B, T, D)
